# Initial kernel scaffold; baseline (speedup 1.0000x reference)
import jax
import jax.numpy as jnp
from jax.experimental import pallas as pl


def kernel(hidden_states, gate_w, expert_gate_up, expert_down, shared_gate_up, shared_down):
    raise NotImplementedError("write your pallas kernel here")


# speedup vs baseline: 1.3752x; 1.3752x over previous
import jax
import jax.numpy as jnp
from jax.experimental import pallas as pl
from jax.experimental.pallas import tpu as pltpu

_E = 16
_D = 768
_FF = 384
_T = 2048


def _mm_t(a, b):
    return jax.lax.dot_general(
        a, b, (((1,), (1,)), ((), ())), preferred_element_type=jnp.float32
    )


def _silu(g):
    return g * (1.0 / (1.0 + jnp.exp(-g)))


def _moe_body(x_ref, gw_ref, wgu_ref, wd_ref, sgu_ref, sd_ref, out_ref):
    e = pl.program_id(0)
    x = x_ref[...]

    logits = _mm_t(x, gw_ref[...])
    iota = jax.lax.broadcasted_iota(jnp.int32, (_T, _E), 1)
    m1 = jnp.max(logits, axis=-1, keepdims=True)
    is1 = logits == m1
    j1 = jnp.min(jnp.where(is1, iota, _E), axis=-1, keepdims=True)
    first1 = iota == j1
    rest = jnp.where(first1, -jnp.inf, logits)
    m2 = jnp.max(rest, axis=-1, keepdims=True)
    is2 = rest == m2
    j2 = jnp.min(jnp.where(is2, iota, _E), axis=-1, keepdims=True)
    sel = first1 | (iota == j2)
    ex = jnp.where(sel, jnp.exp(logits - m1), 0.0)
    denom = jnp.sum(ex, axis=-1, keepdims=True)
    col = jnp.sum(jnp.where(iota == e, ex, 0.0), axis=-1, keepdims=True) / denom

    gu = _mm_t(x, wgu_ref[0])
    act = _silu(gu[:, :_FF]) * gu[:, _FF:]
    dn = _mm_t(act, wd_ref[0])
    contrib = dn * col

    @pl.when(e == 0)
    def _():
        sgu = _mm_t(x, sgu_ref[...])
        sact = _silu(sgu[:, :_FF]) * sgu[:, _FF:]
        out_ref[...] = _mm_t(sact, sd_ref[...]) + contrib

    @pl.when(e != 0)
    def _():
        out_ref[...] += contrib


def kernel(hidden_states, gate_w, expert_gate_up, expert_down,
           shared_gate_up, shared_down):
    return pl.pallas_call(
        _moe_body,
        grid=(_E,),
        in_specs=[
            pl.BlockSpec((_T, _D), lambda e: (0, 0)),
            pl.BlockSpec((_E, _D), lambda e: (0, 0)),
            pl.BlockSpec((1, 2 * _FF, _D), lambda e: (e, 0, 0)),
            pl.BlockSpec((1, _D, _FF), lambda e: (e, 0, 0)),
            pl.BlockSpec((2 * _FF, _D), lambda e: (0, 0)),
            pl.BlockSpec((_D, _FF), lambda e: (0, 0)),
        ],
        out_specs=pl.BlockSpec((_T, _D), lambda e: (0, 0)),
        out_shape=jax.ShapeDtypeStruct((_T, _D), jnp.float32),
        compiler_params=pltpu.CompilerParams(
            dimension_semantics=("arbitrary",),
        ),
    )(hidden_states, gate_w, expert_gate_up, expert_down,
      shared_gate_up, shared_down)


# speedup vs baseline: 1.3816x; 1.0047x over previous
import jax
import jax.numpy as jnp
from jax.experimental import pallas as pl
from jax.experimental.pallas import tpu as pltpu

_E = 16
_D = 768
_FF = 384
_T = 2048


def _mm_t(a, b):
    return jax.lax.dot_general(
        a, b, (((1,), (1,)), ((), ())), preferred_element_type=jnp.float32
    )


def _mm_t_bf16(a, b):
    return jax.lax.dot_general(
        a.astype(jnp.bfloat16), b.astype(jnp.bfloat16),
        (((1,), (1,)), ((), ())), preferred_element_type=jnp.float32,
    )


def _silu(g):
    return g * (1.0 / (1.0 + jnp.exp(-g)))


def _moe_body(x_ref, gw_ref, wgu_ref, wd_ref, sgu_ref, sd_ref, out_ref):
    e = pl.program_id(0)
    x = x_ref[...]

    logits = _mm_t(x, gw_ref[...])
    iota = jax.lax.broadcasted_iota(jnp.int32, (_T, _E), 1)
    m1 = jnp.max(logits, axis=-1, keepdims=True)
    is1 = logits == m1
    j1 = jnp.min(jnp.where(is1, iota, _E), axis=-1, keepdims=True)
    first1 = iota == j1
    rest = jnp.where(first1, -jnp.inf, logits)
    m2 = jnp.max(rest, axis=-1, keepdims=True)
    is2 = rest == m2
    j2 = jnp.min(jnp.where(is2, iota, _E), axis=-1, keepdims=True)
    sel = first1 | (iota == j2)
    ex = jnp.where(sel, jnp.exp(logits - m1), 0.0)
    denom = jnp.sum(ex, axis=-1, keepdims=True)
    col = jnp.sum(jnp.where(iota == e, ex, 0.0), axis=-1, keepdims=True) / denom

    gu = _mm_t_bf16(x, wgu_ref[0])
    act = _silu(gu[:, :_FF]) * gu[:, _FF:]
    dn = _mm_t_bf16(act, wd_ref[0])
    contrib = dn * col

    @pl.when(e == 0)
    def _():
        sgu = _mm_t_bf16(x, sgu_ref[...])
        sact = _silu(sgu[:, :_FF]) * sgu[:, _FF:]
        out_ref[...] = _mm_t_bf16(sact, sd_ref[...]) + contrib

    @pl.when(e != 0)
    def _():
        out_ref[...] += contrib


def kernel(hidden_states, gate_w, expert_gate_up, expert_down,
           shared_gate_up, shared_down):
    return pl.pallas_call(
        _moe_body,
        grid=(_E,),
        in_specs=[
            pl.BlockSpec((_T, _D), lambda e: (0, 0)),
            pl.BlockSpec((_E, _D), lambda e: (0, 0)),
            pl.BlockSpec((1, 2 * _FF, _D), lambda e: (e, 0, 0)),
            pl.BlockSpec((1, _D, _FF), lambda e: (e, 0, 0)),
            pl.BlockSpec((2 * _FF, _D), lambda e: (0, 0)),
            pl.BlockSpec((_D, _FF), lambda e: (0, 0)),
        ],
        out_specs=pl.BlockSpec((_T, _D), lambda e: (0, 0)),
        out_shape=jax.ShapeDtypeStruct((_T, _D), jnp.float32),
        compiler_params=pltpu.CompilerParams(
            dimension_semantics=("arbitrary",),
        ),
    )(hidden_states, gate_w, expert_gate_up, expert_down,
      shared_gate_up, shared_down)


# speedup vs baseline: 1.5716x; 1.1375x over previous
import jax
import jax.numpy as jnp
from jax.experimental import pallas as pl
from jax.experimental.pallas import tpu as pltpu

_E = 16
_D = 768
_FF = 384
_T = 2048


def _mm_t(a, b):
    return jax.lax.dot_general(
        a, b, (((1,), (1,)), ((), ())), preferred_element_type=jnp.float32
    )


def _mm_t_bf16(a, b):
    return jax.lax.dot_general(
        a.astype(jnp.bfloat16), b.astype(jnp.bfloat16),
        (((1,), (1,)), ((), ())), preferred_element_type=jnp.float32,
    )


def _silu(g):
    return g * (1.0 / (1.0 + jnp.exp(-g)))


def _moe_body(x_ref, gw_ref, wgu_ref, wd_ref, sgu_ref, sd_ref, out_ref,
              comb_ref, xb_ref):
    e = pl.program_id(0)
    iota = jax.lax.broadcasted_iota(jnp.int32, (_T, _E), 1)

    @pl.when(e == 0)
    def _():
        x = x_ref[...]
        xb_ref[...] = x.astype(jnp.bfloat16)
        logits = _mm_t(x, gw_ref[...])
        m1 = jnp.max(logits, axis=-1, keepdims=True)
        is1 = logits == m1
        j1 = jnp.min(jnp.where(is1, iota, _E), axis=-1, keepdims=True)
        first1 = iota == j1
        rest = jnp.where(first1, -jnp.inf, logits)
        m2 = jnp.max(rest, axis=-1, keepdims=True)
        is2 = rest == m2
        j2 = jnp.min(jnp.where(is2, iota, _E), axis=-1, keepdims=True)
        sel = first1 | (iota == j2)
        ex = jnp.where(sel, jnp.exp(logits - m1), 0.0)
        denom = jnp.sum(ex, axis=-1, keepdims=True)
        comb_ref[...] = ex / denom

    xb = xb_ref[...]
    col = jnp.sum(jnp.where(iota == e, comb_ref[...], 0.0),
                  axis=-1, keepdims=True)

    gu = _mm_t_bf16(xb, wgu_ref[0])
    act = _silu(gu[:, :_FF]) * gu[:, _FF:]
    dn = _mm_t_bf16(act, wd_ref[0])
    contrib = dn * col

    @pl.when(e == 0)
    def _():
        sgu = _mm_t_bf16(xb, sgu_ref[...])
        sact = _silu(sgu[:, :_FF]) * sgu[:, _FF:]
        out_ref[...] = _mm_t_bf16(sact, sd_ref[...]) + contrib

    @pl.when(e != 0)
    def _():
        out_ref[...] += contrib


def kernel(hidden_states, gate_w, expert_gate_up, expert_down,
           shared_gate_up, shared_down):
    return pl.pallas_call(
        _moe_body,
        grid=(_E,),
        in_specs=[
            pl.BlockSpec((_T, _D), lambda e: (0, 0)),
            pl.BlockSpec((_E, _D), lambda e: (0, 0)),
            pl.BlockSpec((1, 2 * _FF, _D), lambda e: (e, 0, 0)),
            pl.BlockSpec((1, _D, _FF), lambda e: (e, 0, 0)),
            pl.BlockSpec((2 * _FF, _D), lambda e: (0, 0)),
            pl.BlockSpec((_D, _FF), lambda e: (0, 0)),
        ],
        out_specs=pl.BlockSpec((_T, _D), lambda e: (0, 0)),
        out_shape=jax.ShapeDtypeStruct((_T, _D), jnp.float32),
        scratch_shapes=[
            pltpu.VMEM((_T, _E), jnp.float32),
            pltpu.VMEM((_T, _D), jnp.bfloat16),
        ],
        compiler_params=pltpu.CompilerParams(
            dimension_semantics=("arbitrary",),
        ),
    )(hidden_states, gate_w, expert_gate_up, expert_down,
      shared_gate_up, shared_down)
